# baseline (speedup 1.0000x reference)
import functools

import jax
import jax.numpy as jnp
from jax import lax
from jax.experimental import pallas as pl
from jax.experimental.pallas import tpu as pltpu
from jax.experimental.pallas import tpu_sc as plsc

BITS = 4096
LENGTH = 1024
NMAP = 10

NT1 = 16
NPT1 = BITS // NT1
G1 = NPT1 // 16

NW2 = 32
NPT2 = BITS // NW2
G2 = NPT2 // 16


@functools.partial(
    pl.kernel,
    out_type=jax.ShapeDtypeStruct((LENGTH, BITS), jnp.int32),
    mesh=plsc.VectorSubcoreMesh(
        core_axis_name="c", subcore_axis_name="s", num_cores=1
    ),
    compiler_params=pltpu.CompilerParams(needs_layout_passes=False, use_tc_tiling_on_sc=False),
    scratch_types=[
        pltpu.VMEM((BITS + 16,), jnp.int32),
        pltpu.VMEM((NMAP, NPT1), jnp.int32),
        pltpu.VMEM((NPT1, 32), jnp.int32),
        pltpu.VMEM((LENGTH * 16,), jnp.int32),
        pltpu.VMEM((2, 4, NPT1), jnp.int32),
        pltpu.VMEM((NPT1,), jnp.int32),
        pltpu.VMEM_SHARED((2 * BITS,), jnp.int32),
        pltpu.SemaphoreType.DMA,
        pltpu.SemaphoreType.DMA,
    ],
)
def _scan_kernel(tmapT_hbm, tbw_hbm, posb_hbm, bits0_hbm, addr_hbm,
                 inp_v, map_v, tbw_v, posb_v, stage_v, newb_v, shared_s,
                 sem, psem):
    t = lax.axis_index("s")
    n0 = t * NPT1
    iota16 = lax.iota(jnp.int32, 16)

    pltpu.sync_copy(tmapT_hbm.at[:, pl.ds(n0, NPT1)], map_v)
    pltpu.sync_copy(tbw_hbm.at[pl.ds(n0, NPT1), :], tbw_v)
    pltpu.sync_copy(posb_hbm, posb_v)
    pltpu.sync_copy(bits0_hbm, inp_v.at[pl.ds(0, BITS)])

    zero16 = jnp.zeros((16,), jnp.int32)
    for g in range(G1):
        stage_v[0, 0, pl.ds(g * 16, 16)] = zero16

    def _step(i, carry):
        inp_v[pl.ds(BITS, 16)] = posb_v[pl.ds(i * 16, 16)]

        slot = lax.shift_right_logical(i, 2) & 1
        sub = i & 3

        for g in range(G1):
            addr = jnp.zeros((16,), jnp.int32)
            for j in range(NMAP):
                idx = map_v[j, pl.ds(g * 16, 16)]
                b = plsc.load_gather(inp_v, [idx])
                addr = addr + lax.shift_left(b, jnp.int32(j))
            row = g * 16 + iota16
            w = plsc.load_gather(tbw_v, [row, lax.shift_right_logical(addr, 5)])
            bit = lax.shift_right_logical(w, addr & 31) & 1
            newb_v[pl.ds(g * 16, 16)] = bit
            stage_v[slot, sub, pl.ds(g * 16, 16)] = addr

        par = i & 1
        pub = pltpu.async_copy(
            newb_v, shared_s.at[pl.ds(par * BITS + n0, NPT1)], psem)

        @pl.when(((i & 3) == 0) & (i >= 8))
        def _():
            pltpu.make_async_copy(
                stage_v.at[slot],
                addr_hbm.at[pl.ds(i - 8, 4), pl.ds(n0, NPT1)],
                sem,
            ).wait()
        @pl.when((i & 3) == 3)
        def _():
            pltpu.async_copy(
                stage_v.at[slot],
                addr_hbm.at[pl.ds(i - 3, 4), pl.ds(n0, NPT1)],
                sem,
            )

        pub.wait()
        plsc.subcore_barrier()
        pltpu.sync_copy(shared_s.at[pl.ds(par * BITS, BITS)],
                        inp_v.at[pl.ds(0, BITS)])
        return carry

    lax.fori_loop(1, LENGTH, _step, 0)

    for k in range(2):
        i0 = LENGTH - 8 + 4 * k
        pltpu.make_async_copy(
            stage_v.at[(i0 >> 2) & 1],
            addr_hbm.at[pl.ds(i0, 4), pl.ds(n0, NPT1)],
            sem,
        ).wait()


@functools.partial(
    pl.kernel,
    out_type=jax.ShapeDtypeStruct((LENGTH, BITS), jnp.float32),
    mesh=plsc.VectorSubcoreMesh(core_axis_name="c", subcore_axis_name="s"),
    compiler_params=pltpu.CompilerParams(needs_layout_passes=False, use_tc_tiling_on_sc=False),
    scratch_types=[
        pltpu.VMEM((16, LENGTH), jnp.float32),
        pltpu.VMEM((LENGTH, 16), jnp.int32),
        pltpu.VMEM((LENGTH, 16), jnp.float32),
        pltpu.VMEM((NPT2,), jnp.float32),
    ],
)
def _gather_kernel(tt_hbm, addr_hbm, initc_hbm, out_hbm,
                   rows_v, ablk_v, oblk_v, init_v):
    c = lax.axis_index("c")
    s = lax.axis_index("s")
    w = c * 16 + s
    n0 = w * NPT2
    iota16 = lax.iota(jnp.int32, 16)

    for g in range(G2):
        gn0 = n0 + g * 16
        pltpu.sync_copy(tt_hbm.at[pl.ds(gn0, 16), :], rows_v)
        pltpu.sync_copy(addr_hbm.at[:, pl.ds(gn0, 16)], ablk_v)

        def _row(i, carry):
            ivec = jnp.full((16,), 0, jnp.int32) + i
            a = plsc.load_gather(ablk_v, [ivec, iota16])
            v = plsc.load_gather(rows_v, [iota16, a])
            plsc.store_scatter(oblk_v, [ivec, iota16], v)
            return carry
        lax.fori_loop(0, LENGTH, _row, 0)

        pltpu.sync_copy(oblk_v, out_hbm.at[:, pl.ds(gn0, 16)])

    pltpu.sync_copy(initc_hbm.at[pl.ds(n0, NPT2)], init_v)
    pltpu.sync_copy(init_v, out_hbm.at[0, pl.ds(n0, NPT2)])


def kernel(length, transition_table, initial_table, transition_map, initial_map):
    length = jnp.asarray(length, dtype=jnp.int32)
    del initial_map

    initcol = initial_table[:, 0]
    bits0 = (initcol > 0.5).astype(jnp.int32)
    tmap_t = transition_map.T.astype(jnp.int32)

    tb = (transition_table > 0.5).astype(jnp.uint32).reshape(BITS, 32, 32)
    tbw = lax.bitcast_convert_type(
        jnp.sum(tb << jnp.arange(32, dtype=jnp.uint32)[None, None, :], axis=2),
        jnp.int32,
    )

    pos = jnp.arange(LENGTH, dtype=jnp.int32) + (length - LENGTH)
    k = jnp.arange(16, dtype=jnp.int32)
    posb = jnp.where(
        k[None, :] < 4,
        (pos[:, None] >> (3 - jnp.minimum(k, 3))[None, :]) & 1,
        0,
    ).astype(jnp.int32).reshape(-1)

    addr = _scan_kernel(tmap_t, tbw, posb, bits0)
    return _gather_kernel(transition_table, addr, initcol)


# speedup vs baseline: 1.2691x; 1.2691x over previous
import functools

import jax
import jax.numpy as jnp
from jax import lax
from jax.experimental import pallas as pl
from jax.experimental.pallas import tpu as pltpu
from jax.experimental.pallas import tpu_sc as plsc

BITS = 4096
LENGTH = 1024
NMAP = 10

NT1 = 16
NPT1 = BITS // NT1
G1 = NPT1 // 16

NW2 = 32
NPT2 = BITS // NW2
G2 = NPT2 // 16


@functools.partial(
    pl.kernel,
    out_type=jax.ShapeDtypeStruct((LENGTH, BITS), jnp.int32),
    mesh=plsc.VectorSubcoreMesh(
        core_axis_name="c", subcore_axis_name="s", num_cores=1
    ),
    compiler_params=pltpu.CompilerParams(needs_layout_passes=False, use_tc_tiling_on_sc=False),
    scratch_types=[
        pltpu.VMEM((BITS + 16,), jnp.int32),
        pltpu.VMEM((NMAP, NPT1), jnp.int32),
        pltpu.VMEM((NPT1, 32), jnp.int32),
        pltpu.VMEM((LENGTH * 16,), jnp.int32),
        pltpu.VMEM((2, 4, NPT1), jnp.int32),
        pltpu.VMEM((NPT1,), jnp.int32),
        pltpu.VMEM_SHARED((2 * BITS,), jnp.int32),
        pltpu.SemaphoreType.DMA,
        pltpu.SemaphoreType.DMA,
    ],
)
def _scan_kernel(tmapT_hbm, tbw_hbm, posb_hbm, bits0_hbm, addr_hbm,
                 inp_v, map_v, tbw_v, posb_v, stage_v, newb_v, shared_s,
                 sem, psem):
    t = lax.axis_index("s")
    n0 = t * NPT1
    iota16 = lax.iota(jnp.int32, 16)

    pltpu.sync_copy(tmapT_hbm.at[:, pl.ds(n0, NPT1)], map_v)
    pltpu.sync_copy(tbw_hbm.at[pl.ds(n0, NPT1), :], tbw_v)
    pltpu.sync_copy(posb_hbm, posb_v)
    pltpu.sync_copy(bits0_hbm, inp_v.at[pl.ds(0, BITS)])

    zero16 = jnp.zeros((16,), jnp.int32)
    for g in range(G1):
        stage_v[0, 0, pl.ds(g * 16, 16)] = zero16

    def _step(i, carry):
        inp_v[pl.ds(BITS, 16)] = posb_v[pl.ds(i * 16, 16)]

        slot = lax.shift_right_logical(i, 2) & 1
        sub = i & 3

        @plsc.parallel_loop(0, G1, unroll=4)
        def _grp(g):
            addr = jnp.zeros((16,), jnp.int32)
            for j in range(NMAP):
                idx = map_v[j, pl.ds(g * 16, 16)]
                b = plsc.load_gather(inp_v, [idx])
                addr = addr + lax.shift_left(b, jnp.int32(j))
            row = g * 16 + iota16
            w = plsc.load_gather(tbw_v, [row, lax.shift_right_logical(addr, 5)])
            bit = lax.shift_right_logical(w, addr & 31) & 1
            newb_v[pl.ds(g * 16, 16)] = bit
            stage_v[slot, sub, pl.ds(g * 16, 16)] = addr

        par = i & 1
        pub = pltpu.async_copy(
            newb_v, shared_s.at[pl.ds(par * BITS + n0, NPT1)], psem)

        @pl.when(((i & 3) == 0) & (i >= 8))
        def _():
            pltpu.make_async_copy(
                stage_v.at[slot],
                addr_hbm.at[pl.ds(i - 8, 4), pl.ds(n0, NPT1)],
                sem,
            ).wait()
        @pl.when((i & 3) == 3)
        def _():
            pltpu.async_copy(
                stage_v.at[slot],
                addr_hbm.at[pl.ds(i - 3, 4), pl.ds(n0, NPT1)],
                sem,
            )

        pub.wait()
        plsc.subcore_barrier()
        pltpu.sync_copy(shared_s.at[pl.ds(par * BITS, BITS)],
                        inp_v.at[pl.ds(0, BITS)])
        return carry

    lax.fori_loop(1, LENGTH, _step, 0)

    for k in range(2):
        i0 = LENGTH - 8 + 4 * k
        pltpu.make_async_copy(
            stage_v.at[(i0 >> 2) & 1],
            addr_hbm.at[pl.ds(i0, 4), pl.ds(n0, NPT1)],
            sem,
        ).wait()


@functools.partial(
    pl.kernel,
    out_type=jax.ShapeDtypeStruct((LENGTH, BITS), jnp.float32),
    mesh=plsc.VectorSubcoreMesh(core_axis_name="c", subcore_axis_name="s"),
    compiler_params=pltpu.CompilerParams(needs_layout_passes=False, use_tc_tiling_on_sc=False),
    scratch_types=[
        pltpu.VMEM((16, LENGTH), jnp.float32),
        pltpu.VMEM((LENGTH, 16), jnp.int32),
        pltpu.VMEM((LENGTH, 16), jnp.float32),
        pltpu.VMEM((NPT2,), jnp.float32),
    ],
)
def _gather_kernel(tt_hbm, addr_hbm, initc_hbm, out_hbm,
                   rows_v, ablk_v, oblk_v, init_v):
    c = lax.axis_index("c")
    s = lax.axis_index("s")
    w = c * 16 + s
    n0 = w * NPT2
    iota16 = lax.iota(jnp.int32, 16)

    for g in range(G2):
        gn0 = n0 + g * 16
        pltpu.sync_copy(tt_hbm.at[pl.ds(gn0, 16), :], rows_v)
        pltpu.sync_copy(addr_hbm.at[:, pl.ds(gn0, 16)], ablk_v)

        def _row(i, carry):
            ivec = jnp.full((16,), 0, jnp.int32) + i
            a = plsc.load_gather(ablk_v, [ivec, iota16])
            v = plsc.load_gather(rows_v, [iota16, a])
            plsc.store_scatter(oblk_v, [ivec, iota16], v)
            return carry
        lax.fori_loop(0, LENGTH, _row, 0)

        pltpu.sync_copy(oblk_v, out_hbm.at[:, pl.ds(gn0, 16)])

    pltpu.sync_copy(initc_hbm.at[pl.ds(n0, NPT2)], init_v)
    pltpu.sync_copy(init_v, out_hbm.at[0, pl.ds(n0, NPT2)])


def kernel(length, transition_table, initial_table, transition_map, initial_map):
    length = jnp.asarray(length, dtype=jnp.int32)
    del initial_map

    initcol = initial_table[:, 0]
    bits0 = (initcol > 0.5).astype(jnp.int32)
    tmap_t = transition_map.T.astype(jnp.int32)

    tb = (transition_table > 0.5).astype(jnp.uint32).reshape(BITS, 32, 32)
    tbw = lax.bitcast_convert_type(
        jnp.sum(tb << jnp.arange(32, dtype=jnp.uint32)[None, None, :], axis=2),
        jnp.int32,
    )

    pos = jnp.arange(LENGTH, dtype=jnp.int32) + (length - LENGTH)
    k = jnp.arange(16, dtype=jnp.int32)
    posb = jnp.where(
        k[None, :] < 4,
        (pos[:, None] >> (3 - jnp.minimum(k, 3))[None, :]) & 1,
        0,
    ).astype(jnp.int32).reshape(-1)

    addr = _scan_kernel(tmap_t, tbw, posb, bits0)
    return _gather_kernel(transition_table, addr, initcol)


# speedup vs baseline: 1.3681x; 1.0780x over previous
import functools

import jax
import jax.numpy as jnp
from jax import lax
from jax.experimental import pallas as pl
from jax.experimental.pallas import tpu as pltpu
from jax.experimental.pallas import tpu_sc as plsc

BITS = 4096
LENGTH = 1024
NMAP = 10

NT1 = 16
NPT1 = BITS // NT1
G1 = NPT1 // 16

NW2 = 32
NPT2 = BITS // NW2
G2 = NPT2 // 16


@functools.partial(
    pl.kernel,
    out_type=jax.ShapeDtypeStruct((LENGTH, BITS), jnp.int32),
    mesh=plsc.VectorSubcoreMesh(
        core_axis_name="c", subcore_axis_name="s", num_cores=1
    ),
    compiler_params=pltpu.CompilerParams(needs_layout_passes=False, use_tc_tiling_on_sc=False),
    scratch_types=[
        pltpu.VMEM((BITS + 16,), jnp.int32),
        pltpu.VMEM((NMAP, NPT1), jnp.int32),
        pltpu.VMEM((NPT1, 32), jnp.int32),
        pltpu.VMEM((LENGTH * 16,), jnp.int32),
        pltpu.VMEM((2, 4, NPT1), jnp.int32),
        pltpu.VMEM((NPT1,), jnp.int32),
        pltpu.VMEM_SHARED((2 * BITS,), jnp.int32),
        pltpu.SemaphoreType.DMA,
        pltpu.SemaphoreType.DMA,
    ],
)
def _scan_kernel(tmapT_hbm, tbw_hbm, posb_hbm, bits0_hbm, addr_hbm,
                 inp_v, map_v, tbw_v, posb_v, stage_v, newb_v, shared_s,
                 sem, psem):
    t = lax.axis_index("s")
    n0 = t * NPT1
    iota16 = lax.iota(jnp.int32, 16)

    pltpu.sync_copy(tmapT_hbm.at[:, pl.ds(n0, NPT1)], map_v)
    pltpu.sync_copy(tbw_hbm.at[pl.ds(n0, NPT1), :], tbw_v)
    pltpu.sync_copy(posb_hbm, posb_v)
    pltpu.sync_copy(bits0_hbm, inp_v.at[pl.ds(0, BITS)])

    zero16 = jnp.zeros((16,), jnp.int32)
    for g in range(G1):
        stage_v[0, 0, pl.ds(g * 16, 16)] = zero16

    def _step(i, carry):
        inp_v[pl.ds(BITS, 16)] = posb_v[pl.ds(i * 16, 16)]

        slot = lax.shift_right_logical(i, 2) & 1
        sub = i & 3

        @plsc.parallel_loop(0, G1, unroll=8)
        def _grp(g):
            addr = jnp.zeros((16,), jnp.int32)
            for j in range(NMAP):
                idx = map_v[j, pl.ds(g * 16, 16)]
                b = plsc.load_gather(inp_v, [idx])
                addr = addr + lax.shift_left(b, jnp.int32(j))
            row = g * 16 + iota16
            w = plsc.load_gather(tbw_v, [row, lax.shift_right_logical(addr, 5)])
            bit = lax.shift_right_logical(w, addr & 31) & 1
            newb_v[pl.ds(g * 16, 16)] = bit
            stage_v[slot, sub, pl.ds(g * 16, 16)] = addr

        par = i & 1
        pub = pltpu.async_copy(
            newb_v, shared_s.at[pl.ds(par * BITS + n0, NPT1)], psem)

        @pl.when(((i & 3) == 0) & (i >= 8))
        def _():
            pltpu.make_async_copy(
                stage_v.at[slot],
                addr_hbm.at[pl.ds(i - 8, 4), pl.ds(n0, NPT1)],
                sem,
            ).wait()
        @pl.when((i & 3) == 3)
        def _():
            pltpu.async_copy(
                stage_v.at[slot],
                addr_hbm.at[pl.ds(i - 3, 4), pl.ds(n0, NPT1)],
                sem,
            )

        pub.wait()
        plsc.subcore_barrier()
        pltpu.sync_copy(shared_s.at[pl.ds(par * BITS, BITS)],
                        inp_v.at[pl.ds(0, BITS)])
        return carry

    lax.fori_loop(1, LENGTH, _step, 0)

    for k in range(2):
        i0 = LENGTH - 8 + 4 * k
        pltpu.make_async_copy(
            stage_v.at[(i0 >> 2) & 1],
            addr_hbm.at[pl.ds(i0, 4), pl.ds(n0, NPT1)],
            sem,
        ).wait()


@functools.partial(
    pl.kernel,
    out_type=jax.ShapeDtypeStruct((LENGTH, BITS), jnp.float32),
    mesh=plsc.VectorSubcoreMesh(core_axis_name="c", subcore_axis_name="s"),
    compiler_params=pltpu.CompilerParams(needs_layout_passes=False, use_tc_tiling_on_sc=False),
    scratch_types=[
        pltpu.VMEM((16, LENGTH), jnp.float32),
        pltpu.VMEM((LENGTH, 16), jnp.int32),
        pltpu.VMEM((LENGTH, 16), jnp.float32),
        pltpu.VMEM((NPT2,), jnp.float32),
    ],
)
def _gather_kernel(tt_hbm, addr_hbm, initc_hbm, out_hbm,
                   rows_v, ablk_v, oblk_v, init_v):
    c = lax.axis_index("c")
    s = lax.axis_index("s")
    w = c * 16 + s
    n0 = w * NPT2
    iota16 = lax.iota(jnp.int32, 16)

    for g in range(G2):
        gn0 = n0 + g * 16
        pltpu.sync_copy(tt_hbm.at[pl.ds(gn0, 16), :], rows_v)
        pltpu.sync_copy(addr_hbm.at[:, pl.ds(gn0, 16)], ablk_v)

        @plsc.parallel_loop(0, LENGTH, unroll=8)
        def _row(i):
            ivec = jnp.full((16,), 0, jnp.int32) + i
            a = plsc.load_gather(ablk_v, [ivec, iota16])
            v = plsc.load_gather(rows_v, [iota16, a])
            plsc.store_scatter(oblk_v, [ivec, iota16], v)

        pltpu.sync_copy(oblk_v, out_hbm.at[:, pl.ds(gn0, 16)])

    pltpu.sync_copy(initc_hbm.at[pl.ds(n0, NPT2)], init_v)
    pltpu.sync_copy(init_v, out_hbm.at[0, pl.ds(n0, NPT2)])


def kernel(length, transition_table, initial_table, transition_map, initial_map):
    length = jnp.asarray(length, dtype=jnp.int32)
    del initial_map

    initcol = initial_table[:, 0]
    bits0 = (initcol > 0.5).astype(jnp.int32)
    tmap_t = transition_map.T.astype(jnp.int32)

    tb = (transition_table > 0.5).astype(jnp.uint32).reshape(BITS, 32, 32)
    tbw = lax.bitcast_convert_type(
        jnp.sum(tb << jnp.arange(32, dtype=jnp.uint32)[None, None, :], axis=2),
        jnp.int32,
    )

    pos = jnp.arange(LENGTH, dtype=jnp.int32) + (length - LENGTH)
    k = jnp.arange(16, dtype=jnp.int32)
    posb = jnp.where(
        k[None, :] < 4,
        (pos[:, None] >> (3 - jnp.minimum(k, 3))[None, :]) & 1,
        0,
    ).astype(jnp.int32).reshape(-1)

    addr = _scan_kernel(tmap_t, tbw, posb, bits0)
    return _gather_kernel(transition_table, addr, initcol)


# speedup vs baseline: 1.3769x; 1.0064x over previous
import functools

import jax
import jax.numpy as jnp
from jax import lax
from jax.experimental import pallas as pl
from jax.experimental.pallas import tpu as pltpu
from jax.experimental.pallas import tpu_sc as plsc

BITS = 4096
LENGTH = 1024
NMAP = 10

NT1 = 16
NPT1 = BITS // NT1
G1 = NPT1 // 16

NW2 = 32
NPT2 = BITS // NW2
G2 = NPT2 // 16


@functools.partial(
    pl.kernel,
    out_type=jax.ShapeDtypeStruct((LENGTH, BITS), jnp.int32),
    mesh=plsc.VectorSubcoreMesh(
        core_axis_name="c", subcore_axis_name="s", num_cores=1
    ),
    compiler_params=pltpu.CompilerParams(needs_layout_passes=False, use_tc_tiling_on_sc=False),
    scratch_types=[
        pltpu.VMEM((BITS + 16,), jnp.int32),
        pltpu.VMEM((NMAP, NPT1), jnp.int32),
        pltpu.VMEM((NPT1, 32), jnp.int32),
        pltpu.VMEM((LENGTH * 16,), jnp.int32),
        pltpu.VMEM((2, 4, NPT1), jnp.int32),
        pltpu.VMEM((NPT1,), jnp.int32),
        pltpu.VMEM_SHARED((2 * BITS,), jnp.int32),
        pltpu.SemaphoreType.DMA,
        pltpu.SemaphoreType.DMA,
    ],
)
def _scan_kernel(tmapT_hbm, tbw_hbm, posb_hbm, bits0_hbm, addr_hbm,
                 inp_v, map_v, tbw_v, posb_v, stage_v, newb_v, shared_s,
                 sem, psem):
    t = lax.axis_index("s")
    n0 = t * NPT1
    iota16 = lax.iota(jnp.int32, 16)

    pltpu.sync_copy(tmapT_hbm.at[:, pl.ds(n0, NPT1)], map_v)
    pltpu.sync_copy(tbw_hbm.at[pl.ds(n0, NPT1), :], tbw_v)
    pltpu.sync_copy(posb_hbm, posb_v)
    pltpu.sync_copy(bits0_hbm, inp_v.at[pl.ds(0, BITS)])

    zero16 = jnp.zeros((16,), jnp.int32)
    for g in range(G1):
        stage_v[0, 0, pl.ds(g * 16, 16)] = zero16

    def _step(i, carry):
        inp_v[pl.ds(BITS, 16)] = posb_v[pl.ds(i * 16, 16)]

        slot = lax.shift_right_logical(i, 2) & 1
        sub = i & 3

        @plsc.parallel_loop(0, G1, unroll=16)
        def _grp(g):
            addr = jnp.zeros((16,), jnp.int32)
            for j in range(NMAP):
                idx = map_v[j, pl.ds(g * 16, 16)]
                b = plsc.load_gather(inp_v, [idx])
                addr = addr + lax.shift_left(b, jnp.int32(j))
            row = g * 16 + iota16
            w = plsc.load_gather(tbw_v, [row, lax.shift_right_logical(addr, 5)])
            bit = lax.shift_right_logical(w, addr & 31) & 1
            newb_v[pl.ds(g * 16, 16)] = bit
            stage_v[slot, sub, pl.ds(g * 16, 16)] = addr

        par = i & 1
        pub = pltpu.async_copy(
            newb_v, shared_s.at[pl.ds(par * BITS + n0, NPT1)], psem)

        @pl.when(((i & 3) == 0) & (i >= 8))
        def _():
            pltpu.make_async_copy(
                stage_v.at[slot],
                addr_hbm.at[pl.ds(i - 8, 4), pl.ds(n0, NPT1)],
                sem,
            ).wait()
        @pl.when((i & 3) == 3)
        def _():
            pltpu.async_copy(
                stage_v.at[slot],
                addr_hbm.at[pl.ds(i - 3, 4), pl.ds(n0, NPT1)],
                sem,
            )

        pub.wait()
        plsc.subcore_barrier()
        pltpu.sync_copy(shared_s.at[pl.ds(par * BITS, BITS)],
                        inp_v.at[pl.ds(0, BITS)])
        return carry

    lax.fori_loop(1, LENGTH, _step, 0)

    for k in range(2):
        i0 = LENGTH - 8 + 4 * k
        pltpu.make_async_copy(
            stage_v.at[(i0 >> 2) & 1],
            addr_hbm.at[pl.ds(i0, 4), pl.ds(n0, NPT1)],
            sem,
        ).wait()


@functools.partial(
    pl.kernel,
    out_type=jax.ShapeDtypeStruct((LENGTH, BITS), jnp.float32),
    mesh=plsc.VectorSubcoreMesh(core_axis_name="c", subcore_axis_name="s"),
    compiler_params=pltpu.CompilerParams(needs_layout_passes=False, use_tc_tiling_on_sc=False),
    scratch_types=[
        pltpu.VMEM((16, LENGTH), jnp.float32),
        pltpu.VMEM((LENGTH, 16), jnp.int32),
        pltpu.VMEM((LENGTH, 16), jnp.float32),
        pltpu.VMEM((NPT2,), jnp.float32),
    ],
)
def _gather_kernel(tt_hbm, addr_hbm, initc_hbm, out_hbm,
                   rows_v, ablk_v, oblk_v, init_v):
    c = lax.axis_index("c")
    s = lax.axis_index("s")
    w = c * 16 + s
    n0 = w * NPT2
    iota16 = lax.iota(jnp.int32, 16)

    for g in range(G2):
        gn0 = n0 + g * 16
        pltpu.sync_copy(tt_hbm.at[pl.ds(gn0, 16), :], rows_v)
        pltpu.sync_copy(addr_hbm.at[:, pl.ds(gn0, 16)], ablk_v)

        @plsc.parallel_loop(0, LENGTH, unroll=16)
        def _row(i):
            ivec = jnp.full((16,), 0, jnp.int32) + i
            a = plsc.load_gather(ablk_v, [ivec, iota16])
            v = plsc.load_gather(rows_v, [iota16, a])
            plsc.store_scatter(oblk_v, [ivec, iota16], v)

        pltpu.sync_copy(oblk_v, out_hbm.at[:, pl.ds(gn0, 16)])

    pltpu.sync_copy(initc_hbm.at[pl.ds(n0, NPT2)], init_v)
    pltpu.sync_copy(init_v, out_hbm.at[0, pl.ds(n0, NPT2)])


def kernel(length, transition_table, initial_table, transition_map, initial_map):
    length = jnp.asarray(length, dtype=jnp.int32)
    del initial_map

    initcol = initial_table[:, 0]
    bits0 = (initcol > 0.5).astype(jnp.int32)
    tmap_t = transition_map.T.astype(jnp.int32)

    tb = (transition_table > 0.5).astype(jnp.uint32).reshape(BITS, 32, 32)
    tbw = lax.bitcast_convert_type(
        jnp.sum(tb << jnp.arange(32, dtype=jnp.uint32)[None, None, :], axis=2),
        jnp.int32,
    )

    pos = jnp.arange(LENGTH, dtype=jnp.int32) + (length - LENGTH)
    k = jnp.arange(16, dtype=jnp.int32)
    posb = jnp.where(
        k[None, :] < 4,
        (pos[:, None] >> (3 - jnp.minimum(k, 3))[None, :]) & 1,
        0,
    ).astype(jnp.int32).reshape(-1)

    addr = _scan_kernel(tmap_t, tbw, posb, bits0)
    return _gather_kernel(transition_table, addr, initcol)


# speedup vs baseline: 1.4302x; 1.0387x over previous
import functools

import jax
import jax.numpy as jnp
from jax import lax
from jax.experimental import pallas as pl
from jax.experimental.pallas import tpu as pltpu
from jax.experimental.pallas import tpu_sc as plsc

BITS = 4096
LENGTH = 1024
NMAP = 10

NT1 = 16
NPT1 = BITS // NT1
G1 = NPT1 // 16

NW2 = 32
NPT2 = BITS // NW2
G2 = NPT2 // 16


@functools.partial(
    pl.kernel,
    out_type=jax.ShapeDtypeStruct((LENGTH, BITS), jnp.int32),
    mesh=plsc.VectorSubcoreMesh(
        core_axis_name="c", subcore_axis_name="s", num_cores=1
    ),
    compiler_params=pltpu.CompilerParams(needs_layout_passes=False, use_tc_tiling_on_sc=False),
    scratch_types=[
        pltpu.VMEM((BITS + 16,), jnp.int32),
        pltpu.VMEM((NMAP, NPT1), jnp.int32),
        pltpu.VMEM((NPT1, 32), jnp.int32),
        pltpu.VMEM((LENGTH * 16,), jnp.int32),
        pltpu.VMEM((2, 4, NPT1), jnp.int32),
        pltpu.VMEM((NPT1,), jnp.int32),
        pltpu.VMEM_SHARED((2 * BITS,), jnp.int32),
        pltpu.SemaphoreType.DMA,
        pltpu.SemaphoreType.DMA,
    ],
)
def _scan_kernel(tmapT_hbm, tbw_hbm, posb_hbm, bits0_hbm, addr_hbm,
                 inp_v, map_v, tbw_v, posb_v, stage_v, newb_v, shared_s,
                 sem, psem):
    t = lax.axis_index("s")
    n0 = t * NPT1
    iota16 = lax.iota(jnp.int32, 16)

    pltpu.sync_copy(tmapT_hbm.at[:, pl.ds(n0, NPT1)], map_v)
    pltpu.sync_copy(tbw_hbm.at[pl.ds(n0, NPT1), :], tbw_v)
    pltpu.sync_copy(posb_hbm, posb_v)
    pltpu.sync_copy(bits0_hbm, inp_v.at[pl.ds(0, BITS)])

    zero16 = jnp.zeros((16,), jnp.int32)
    for g in range(G1):
        stage_v[0, 0, pl.ds(g * 16, 16)] = zero16

    def _step(i, carry):
        inp_v[pl.ds(BITS, 16)] = posb_v[pl.ds(i * 16, 16)]

        slot = lax.shift_right_logical(i, 2) & 1
        sub = i & 3

        @plsc.parallel_loop(0, G1, unroll=16)
        def _grp(g):
            addr = jnp.zeros((16,), jnp.int32)
            for j in range(NMAP):
                idx = map_v[j, pl.ds(g * 16, 16)]
                b = plsc.load_gather(inp_v, [idx])
                addr = addr + lax.shift_left(b, jnp.int32(j))
            row = g * 16 + iota16
            w = plsc.load_gather(tbw_v, [row, lax.shift_right_logical(addr, 5)])
            bit = lax.shift_right_logical(w, addr & 31) & 1
            newb_v[pl.ds(g * 16, 16)] = bit
            stage_v[slot, sub, pl.ds(g * 16, 16)] = addr

        par = i & 1
        pub = pltpu.async_copy(
            newb_v, shared_s.at[pl.ds(par * BITS + n0, NPT1)], psem)

        @pl.when(((i & 3) == 0) & (i >= 8))
        def _():
            pltpu.make_async_copy(
                stage_v.at[slot],
                addr_hbm.at[pl.ds(i - 8, 4), pl.ds(n0, NPT1)],
                sem,
            ).wait()
        @pl.when((i & 3) == 3)
        def _():
            pltpu.async_copy(
                stage_v.at[slot],
                addr_hbm.at[pl.ds(i - 3, 4), pl.ds(n0, NPT1)],
                sem,
            )

        pub.wait()
        plsc.subcore_barrier()
        pltpu.sync_copy(shared_s.at[pl.ds(par * BITS, BITS)],
                        inp_v.at[pl.ds(0, BITS)])
        return carry

    lax.fori_loop(1, LENGTH, _step, 0)

    for k in range(2):
        i0 = LENGTH - 8 + 4 * k
        pltpu.make_async_copy(
            stage_v.at[(i0 >> 2) & 1],
            addr_hbm.at[pl.ds(i0, 4), pl.ds(n0, NPT1)],
            sem,
        ).wait()


@functools.partial(
    pl.kernel,
    out_type=jax.ShapeDtypeStruct((LENGTH, BITS), jnp.float32),
    mesh=plsc.VectorSubcoreMesh(core_axis_name="c", subcore_axis_name="s"),
    compiler_params=pltpu.CompilerParams(needs_layout_passes=False, use_tc_tiling_on_sc=False),
    scratch_types=[
        pltpu.VMEM((2, 16, LENGTH), jnp.float32),
        pltpu.VMEM((2, LENGTH, 16), jnp.int32),
        pltpu.VMEM((2, LENGTH, 16), jnp.float32),
        pltpu.VMEM((NPT2,), jnp.float32),
        pltpu.SemaphoreType.DMA,
        pltpu.SemaphoreType.DMA,
        pltpu.SemaphoreType.DMA,
    ],
)
def _gather_kernel(tt_hbm, addr_hbm, initc_hbm, out_hbm,
                   rows_v, ablk_v, oblk_v, init_v, rsem, asem, osem):
    c = lax.axis_index("c")
    s = lax.axis_index("s")
    w = c * 16 + s
    n0 = w * NPT2
    iota16 = lax.iota(jnp.int32, 16)

    def _fetch(g):
        gn0 = n0 + g * 16
        pltpu.async_copy(tt_hbm.at[pl.ds(gn0, 16), :], rows_v.at[g & 1], rsem)
        pltpu.async_copy(addr_hbm.at[:, pl.ds(gn0, 16)], ablk_v.at[g & 1], asem)

    _fetch(0)
    for g in range(G2):
        gn0 = n0 + g * 16
        b = g & 1
        if g + 1 < G2:
            _fetch(g + 1)
        pltpu.make_async_copy(tt_hbm.at[pl.ds(gn0, 16), :],
                              rows_v.at[b], rsem).wait()
        pltpu.make_async_copy(addr_hbm.at[:, pl.ds(gn0, 16)],
                              ablk_v.at[b], asem).wait()
        if g >= 2:
            pltpu.make_async_copy(
                oblk_v.at[b], out_hbm.at[:, pl.ds(n0 + (g - 2) * 16, 16)],
                osem).wait()

        ab = ablk_v.at[b]
        rb = rows_v.at[b]
        ob = oblk_v.at[b]

        @plsc.parallel_loop(0, LENGTH, unroll=16)
        def _row(i):
            ivec = jnp.full((16,), 0, jnp.int32) + i
            a = plsc.load_gather(ab, [ivec, iota16])
            v = plsc.load_gather(rb, [iota16, a])
            plsc.store_scatter(ob, [ivec, iota16], v)

        pltpu.async_copy(oblk_v.at[b], out_hbm.at[:, pl.ds(gn0, 16)], osem)

    for g in (G2 - 2, G2 - 1):
        pltpu.make_async_copy(
            oblk_v.at[g & 1], out_hbm.at[:, pl.ds(n0 + g * 16, 16)],
            osem).wait()

    pltpu.sync_copy(initc_hbm.at[pl.ds(n0, NPT2)], init_v)
    pltpu.sync_copy(init_v, out_hbm.at[0, pl.ds(n0, NPT2)])


def kernel(length, transition_table, initial_table, transition_map, initial_map):
    length = jnp.asarray(length, dtype=jnp.int32)
    del initial_map

    initcol = initial_table[:, 0]
    bits0 = (initcol > 0.5).astype(jnp.int32)
    tmap_t = transition_map.T.astype(jnp.int32)

    tb = (transition_table > 0.5).astype(jnp.uint32).reshape(BITS, 32, 32)
    tbw = lax.bitcast_convert_type(
        jnp.sum(tb << jnp.arange(32, dtype=jnp.uint32)[None, None, :], axis=2),
        jnp.int32,
    )

    pos = jnp.arange(LENGTH, dtype=jnp.int32) + (length - LENGTH)
    k = jnp.arange(16, dtype=jnp.int32)
    posb = jnp.where(
        k[None, :] < 4,
        (pos[:, None] >> (3 - jnp.minimum(k, 3))[None, :]) & 1,
        0,
    ).astype(jnp.int32).reshape(-1)

    addr = _scan_kernel(tmap_t, tbw, posb, bits0)
    return _gather_kernel(transition_table, addr, initcol)
